# Initial kernel scaffold; baseline (speedup 1.0000x reference)
import jax
import jax.numpy as jnp
from jax.experimental import pallas as pl


def kernel(node_matrix, graph, W1, b1, W2, b2, Wl, bl):
    raise NotImplementedError("write your pallas kernel here")


# speedup vs baseline: 15.2853x; 15.2853x over previous
import functools

import jax
import jax.numpy as jnp
from jax import lax
from jax.experimental import pallas as pl
from jax.experimental.pallas import tpu as pltpu
from jax.experimental.pallas import tpu_sc as plsc

N = 50000
IN_F = 16
H1 = 64
H2 = 128
OUT_F = 16

NC = 2
NS = 16
L = 16
NW = NC * NS

CHUNK = 128
NPAD = 50048
NBLK = NPAD // CHUNK
E_PAD = 819200
ECH = E_PAD // CHUNK
CH1 = ECH // NW
CH2 = ECH // NS
BLK2 = 40
NB2 = CH2 // BLK2
RPT = NPAD // NS
RFULL = RPT // CHUNK
RREM = RPT - RFULL * CHUNK

_mesh = plsc.VectorSubcoreMesh(core_axis_name="c", subcore_axis_name="s")
_sc_params = pltpu.CompilerParams(
    use_tc_tiling_on_sc=False, needs_layout_passes=False)


@functools.partial(
    pl.kernel,
    out_type=jax.ShapeDtypeStruct((NW, NPAD), jnp.float32),
    mesh=_mesh,
    compiler_params=_sc_params,
    scratch_types=[
        pltpu.VMEM((NPAD,), jnp.float32),
        pltpu.VMEM((E_PAD // NW,), jnp.int32),
    ],
)
def _deg_kernel(dst_hbm, degp_out, deg_v, idx_v):
    c = lax.axis_index("c")
    s = lax.axis_index("s")
    wid = s * NC + c
    epw = E_PAD // NW

    zeros = jnp.zeros((L,), jnp.float32)

    def zbody(i, _):
        deg_v[pl.ds(i * L, L)] = zeros
        return 0

    lax.fori_loop(0, NPAD // L, zbody, 0)

    pltpu.sync_copy(dst_hbm.at[pl.ds(wid * epw, epw)], idx_v)

    ones = jnp.full((L,), 1.0, jnp.float32)

    def body(i, _):
        iv = idx_v[pl.ds(i * L, L)]
        plsc.addupdate_scatter(deg_v, [iv], ones)
        return 0

    lax.fori_loop(0, epw // L, body, 0)
    pltpu.sync_copy(deg_v, degp_out.at[wid])


def _dinv_xs_body(degp_ref, x_ref, dinv_ref, xs_ref):
    deg = jnp.sum(degp_ref[...], axis=0) + 1.0
    dv = lax.rsqrt(deg)
    dinv_ref[...] = dv[None, None, :]
    xs_ref[...] = x_ref[...] * dv[:, None]


def _dinv_xs(degp, xpad):
    return pl.pallas_call(
        _dinv_xs_body,
        grid=(NBLK,),
        in_specs=[
            pl.BlockSpec((NW, CHUNK), lambda i: (0, i)),
            pl.BlockSpec((CHUNK, IN_F), lambda i: (i, 0)),
        ],
        out_specs=[
            pl.BlockSpec((1, 1, CHUNK), lambda i: (i, 0, 0)),
            pl.BlockSpec((CHUNK, IN_F), lambda i: (i, 0)),
        ],
        out_shape=[
            jax.ShapeDtypeStruct((NBLK, 1, CHUNK), jnp.float32),
            jax.ShapeDtypeStruct((NPAD, IN_F), jnp.float32),
        ],
        compiler_params=pltpu.CompilerParams(
            dimension_semantics=("arbitrary",)),
    )(degp, xpad)


@functools.partial(
    pl.kernel,
    out_type=jax.ShapeDtypeStruct((NC, NPAD, IN_F), jnp.float32),
    mesh=_mesh,
    compiler_params=_sc_params,
    scratch_types=[
        pltpu.VMEM((CH1, CHUNK), jnp.int32),
        pltpu.VMEM((CH1, CHUNK), jnp.int32),
        pltpu.VMEM((2, CHUNK, IN_F), jnp.float32),
        pltpu.VMEM((CHUNK, IN_F), jnp.float32),
        pltpu.VMEM_SHARED((NPAD, IN_F), jnp.float32),
        pltpu.SemaphoreType.DMA((2,)),
    ],
)
def _pass1_kernel(src_hbm, dst_hbm, xs_hbm, acc_out,
                  sidx, didx, gbuf, zbuf, acc_sh, sem):
    c = lax.axis_index("c")
    s = lax.axis_index("s")
    base = c * (ECH // NC) + s * CH1

    pltpu.sync_copy(src_hbm.at[pl.ds(base, CH1)], sidx)
    pltpu.sync_copy(dst_hbm.at[pl.ds(base, CH1)], didx)

    zeros = jnp.zeros((L,), jnp.float32)

    def zbody(i, _):
        zbuf[i, pl.ds(0, L)] = zeros
        return 0

    lax.fori_loop(0, CHUNK, zbody, 0)
    row0 = s * RPT
    for k in range(RFULL):
        pltpu.sync_copy(zbuf, acc_sh.at[pl.ds(row0 + k * CHUNK, CHUNK)])
    pltpu.sync_copy(zbuf.at[pl.ds(0, RREM)],
                    acc_sh.at[pl.ds(row0 + RFULL * CHUNK, RREM)])
    plsc.subcore_barrier()

    def gstart(j, b):
        pltpu.make_async_copy(
            xs_hbm.at[sidx.at[j]], gbuf.at[b], sem.at[b]).start()

    def gwait(b):
        pltpu.make_async_copy(
            xs_hbm.at[sidx.at[0]], gbuf.at[b], sem.at[b]).wait()

    gstart(0, 0)

    def body(jj, _):
        j0 = 2 * jj
        gstart(j0 + 1, 1)
        gwait(0)
        pltpu.sync_copy(gbuf.at[0], acc_sh.at[didx.at[j0]], add=True)

        @pl.when(jj < CH1 // 2 - 1)
        def _():
            gstart(j0 + 2, 0)

        gwait(1)
        pltpu.sync_copy(gbuf.at[1], acc_sh.at[didx.at[j0 + 1]], add=True)
        return 0

    lax.fori_loop(0, CH1 // 2, body, 0)
    plsc.subcore_barrier()

    for k in range(RFULL):
        pltpu.sync_copy(acc_sh.at[pl.ds(row0 + k * CHUNK, CHUNK)], zbuf)
        pltpu.sync_copy(zbuf, acc_out.at[c, pl.ds(row0 + k * CHUNK, CHUNK)])
    pltpu.sync_copy(acc_sh.at[pl.ds(row0 + RFULL * CHUNK, RREM)],
                    zbuf.at[pl.ds(0, RREM)])
    pltpu.sync_copy(zbuf.at[pl.ds(0, RREM)],
                    acc_out.at[c, pl.ds(row0 + RFULL * CHUNK, RREM)])


def _mid_body(acc1_ref, xs_ref, dinv_ref, w1_ref, b1_ref, hs_ref):
    a = acc1_ref[0] + acc1_ref[1] + xs_ref[...]
    dv = dinv_ref[0, 0, :]
    y1 = a * dv[:, None]
    h = jnp.dot(y1, w1_ref[...], preferred_element_type=jnp.float32,
                precision=lax.Precision.HIGHEST)
    h = jnp.maximum(h + b1_ref[...], 0.0)
    hs = h * dv[:, None]
    hs_ref[0] = hs[:, : H1 // 2]
    hs_ref[1] = hs[:, H1 // 2:]


def _mid(acc1, xs, dinv2d, W1, b1):
    return pl.pallas_call(
        _mid_body,
        grid=(NBLK,),
        in_specs=[
            pl.BlockSpec((NC, CHUNK, IN_F), lambda i: (0, i, 0)),
            pl.BlockSpec((CHUNK, IN_F), lambda i: (i, 0)),
            pl.BlockSpec((1, 1, CHUNK), lambda i: (i, 0, 0)),
            pl.BlockSpec((IN_F, H1), lambda i: (0, 0)),
            pl.BlockSpec((1, H1), lambda i: (0, 0)),
        ],
        out_specs=pl.BlockSpec((NC, CHUNK, H1 // 2), lambda i: (0, i, 0)),
        out_shape=jax.ShapeDtypeStruct((NC, NPAD, H1 // 2), jnp.float32),
        compiler_params=pltpu.CompilerParams(
            dimension_semantics=("arbitrary",)),
    )(acc1, xs, dinv2d, W1, b1)


@functools.partial(
    pl.kernel,
    out_type=jax.ShapeDtypeStruct((NC, NPAD, H1 // 2), jnp.float32),
    mesh=_mesh,
    compiler_params=_sc_params,
    scratch_types=[
        pltpu.VMEM((BLK2, CHUNK), jnp.int32),
        pltpu.VMEM((BLK2, CHUNK), jnp.int32),
        pltpu.VMEM((2, CHUNK, H1 // 2), jnp.float32),
        pltpu.VMEM((CHUNK, H1 // 2), jnp.float32),
        pltpu.VMEM_SHARED((NPAD, H1 // 2), jnp.float32),
        pltpu.SemaphoreType.DMA((2,)),
    ],
)
def _pass2_kernel(src_hbm, dst_hbm, hs_hbm, acc_out,
                  sidx, didx, gbuf, zbuf, acc_sh, sem):
    c = lax.axis_index("c")
    s = lax.axis_index("s")
    base = s * CH2

    zeros = jnp.zeros((L,), jnp.float32)

    def zbody(i, _):
        zbuf[i, pl.ds(0, L)] = zeros
        zbuf[i, pl.ds(L, L)] = zeros
        return 0

    lax.fori_loop(0, CHUNK, zbody, 0)
    row0 = s * RPT
    for k in range(RFULL):
        pltpu.sync_copy(zbuf, acc_sh.at[pl.ds(row0 + k * CHUNK, CHUNK)])
    pltpu.sync_copy(zbuf.at[pl.ds(0, RREM)],
                    acc_sh.at[pl.ds(row0 + RFULL * CHUNK, RREM)])
    plsc.subcore_barrier()

    def gstart(j, b):
        pltpu.make_async_copy(
            hs_hbm.at[c].at[sidx.at[j]], gbuf.at[b], sem.at[b]).start()

    def gwait(b):
        pltpu.make_async_copy(
            hs_hbm.at[c].at[sidx.at[0]], gbuf.at[b], sem.at[b]).wait()

    def blk_body(bb, _):
        pltpu.sync_copy(src_hbm.at[pl.ds(base + bb * BLK2, BLK2)], sidx)
        pltpu.sync_copy(dst_hbm.at[pl.ds(base + bb * BLK2, BLK2)], didx)
        gstart(0, 0)

        def body(jj, _):
            j0 = 2 * jj
            gstart(j0 + 1, 1)
            gwait(0)
            pltpu.sync_copy(gbuf.at[0], acc_sh.at[didx.at[j0]], add=True)

            @pl.when(jj < BLK2 // 2 - 1)
            def _():
                gstart(j0 + 2, 0)

            gwait(1)
            pltpu.sync_copy(gbuf.at[1], acc_sh.at[didx.at[j0 + 1]], add=True)
            return 0

        lax.fori_loop(0, BLK2 // 2, body, 0)
        return 0

    lax.fori_loop(0, NB2, blk_body, 0)
    plsc.subcore_barrier()

    for k in range(RFULL):
        pltpu.sync_copy(acc_sh.at[pl.ds(row0 + k * CHUNK, CHUNK)], zbuf)
        pltpu.sync_copy(zbuf, acc_out.at[c, pl.ds(row0 + k * CHUNK, CHUNK)])
    pltpu.sync_copy(acc_sh.at[pl.ds(row0 + RFULL * CHUNK, RREM)],
                    zbuf.at[pl.ds(0, RREM)])
    pltpu.sync_copy(zbuf.at[pl.ds(0, RREM)],
                    acc_out.at[c, pl.ds(row0 + RFULL * CHUNK, RREM)])


def _out_body(acc2_ref, hs_ref, dinv_ref, w2_ref, wl_ref, b2_ref, bl_ref,
              out_ref):
    dv = dinv_ref[0, 0, :]
    y = jnp.concatenate(
        [acc2_ref[0] + hs_ref[0], acc2_ref[1] + hs_ref[1]], axis=1)
    y = y * dv[:, None]
    wf = jnp.dot(w2_ref[...], wl_ref[...],
                 preferred_element_type=jnp.float32,
                 precision=lax.Precision.HIGHEST)
    bf = jnp.dot(b2_ref[...], wl_ref[...],
                 preferred_element_type=jnp.float32,
                 precision=lax.Precision.HIGHEST) + bl_ref[...]
    out_ref[...] = jnp.dot(y, wf, preferred_element_type=jnp.float32,
                           precision=lax.Precision.HIGHEST) + bf


def _final(acc2, hs, dinv2d, W2, Wl, b2, bl):
    return pl.pallas_call(
        _out_body,
        grid=(NBLK,),
        in_specs=[
            pl.BlockSpec((NC, CHUNK, H1 // 2), lambda i: (0, i, 0)),
            pl.BlockSpec((NC, CHUNK, H1 // 2), lambda i: (0, i, 0)),
            pl.BlockSpec((1, 1, CHUNK), lambda i: (i, 0, 0)),
            pl.BlockSpec((H1, H2), lambda i: (0, 0)),
            pl.BlockSpec((H2, OUT_F), lambda i: (0, 0)),
            pl.BlockSpec((1, H2), lambda i: (0, 0)),
            pl.BlockSpec((1, OUT_F), lambda i: (0, 0)),
        ],
        out_specs=pl.BlockSpec((CHUNK, OUT_F), lambda i: (i, 0)),
        out_shape=jax.ShapeDtypeStruct((NPAD, OUT_F), jnp.float32),
        compiler_params=pltpu.CompilerParams(
            dimension_semantics=("arbitrary",)),
    )(acc2, hs, dinv2d, W2, Wl, b2, bl)


def kernel(node_matrix, graph, W1, b1, W2, b2, Wl, bl):
    E = graph.shape[1]
    src = graph[0]
    dst = graph[1]
    sent = jnp.full((E_PAD - E,), N, jnp.int32)
    srcp = jnp.concatenate([src, sent]).reshape(ECH, CHUNK)
    dstp_flat = jnp.concatenate([dst, sent])
    dstp = dstp_flat.reshape(ECH, CHUNK)

    xpad = jnp.zeros((NPAD, IN_F), jnp.float32).at[:N].set(node_matrix)

    degp = _deg_kernel(dstp_flat)
    dinv2d, xs = _dinv_xs(degp, xpad)
    acc1 = _pass1_kernel(srcp, dstp, xs)
    hs = _mid(acc1, xs, dinv2d, W1, b1.reshape(1, H1))
    acc2 = _pass2_kernel(srcp, dstp, hs)
    outp = _final(acc2, hs, dinv2d, W2, Wl,
                  b2.reshape(1, H2), bl.reshape(1, OUT_F))
    return outp[:N]


# speedup vs baseline: 19.4639x; 1.2734x over previous
import functools

import jax
import jax.numpy as jnp
from jax import lax
from jax.experimental import pallas as pl
from jax.experimental.pallas import tpu as pltpu
from jax.experimental.pallas import tpu_sc as plsc

N = 50000
IN_F = 16
H1 = 64
H2 = 128
OUT_F = 16

NC = 2
NS = 16
L = 16
NW = NC * NS

CHUNK = 128
NPAD = 50048
NBLK = NPAD // CHUNK
E_PAD = 819200
ECH = E_PAD // CHUNK
CH1 = ECH // NW
CH2 = ECH // NS
BLK2 = 40
NB2 = CH2 // BLK2
RPT = NPAD // NS
RFULL = RPT // CHUNK
RREM = RPT - RFULL * CHUNK

_mesh = plsc.VectorSubcoreMesh(core_axis_name="c", subcore_axis_name="s")
_sc_params = pltpu.CompilerParams(
    use_tc_tiling_on_sc=False, needs_layout_passes=False)


@functools.partial(
    pl.kernel,
    out_type=jax.ShapeDtypeStruct((NW, NPAD), jnp.float32),
    mesh=_mesh,
    compiler_params=_sc_params,
    scratch_types=[
        pltpu.VMEM((NPAD,), jnp.float32),
        pltpu.VMEM((E_PAD // NW,), jnp.int32),
    ],
)
def _deg_kernel(dst_hbm, degp_out, deg_v, idx_v):
    c = lax.axis_index("c")
    s = lax.axis_index("s")
    wid = s * NC + c
    epw = E_PAD // NW

    zeros = jnp.zeros((L,), jnp.float32)

    def zbody(i, _):
        deg_v[pl.ds(i * L, L)] = zeros
        return 0

    lax.fori_loop(0, NPAD // L, zbody, 0)

    pltpu.sync_copy(dst_hbm.at[pl.ds(wid * epw, epw)], idx_v)

    ones = jnp.full((L,), 1.0, jnp.float32)

    def body(i, _):
        iv = idx_v[pl.ds(i * L, L)]
        plsc.addupdate_scatter(deg_v, [iv], ones)
        return 0

    lax.fori_loop(0, epw // L, body, 0)
    pltpu.sync_copy(deg_v, degp_out.at[wid])


def _dinv_xs_body(degp_ref, x_ref, dinv_ref, xs_ref):
    deg = jnp.sum(degp_ref[...], axis=0) + 1.0
    dv = lax.rsqrt(deg)
    dinv_ref[...] = dv[None, None, :]
    xs_ref[...] = x_ref[...] * dv[:, None]


def _dinv_xs(degp, xpad):
    return pl.pallas_call(
        _dinv_xs_body,
        grid=(NBLK,),
        in_specs=[
            pl.BlockSpec((NW, CHUNK), lambda i: (0, i)),
            pl.BlockSpec((CHUNK, IN_F), lambda i: (i, 0)),
        ],
        out_specs=[
            pl.BlockSpec((1, 1, CHUNK), lambda i: (i, 0, 0)),
            pl.BlockSpec((CHUNK, IN_F), lambda i: (i, 0)),
        ],
        out_shape=[
            jax.ShapeDtypeStruct((NBLK, 1, CHUNK), jnp.float32),
            jax.ShapeDtypeStruct((NPAD, IN_F), jnp.float32),
        ],
        compiler_params=pltpu.CompilerParams(
            dimension_semantics=("arbitrary",)),
    )(degp, xpad)


def _run_chunks(nchunks, gstart, sstart, gwait, swait):
    kmax = nchunks // 4
    gstart(0, 0)
    gstart(1, 1)

    def body(kk, _):
        for b in range(4):
            j = 4 * kk + b
            gwait(b)
            sstart(j, b)
            tgt = (b + 2) % 4
            if b < 2:
                @pl.when(kk > 0)
                def _():
                    swait(tgt)

                gstart(j + 2, tgt)
            else:
                @pl.when(kk < kmax - 1)
                def _():
                    swait(tgt)
                    gstart(j + 2, tgt)
        return 0

    lax.fori_loop(0, kmax, body, 0)
    for b in range(4):
        swait(b)


@functools.partial(
    pl.kernel,
    out_type=jax.ShapeDtypeStruct((NC, NPAD, IN_F), jnp.float32),
    mesh=_mesh,
    compiler_params=_sc_params,
    scratch_types=[
        pltpu.VMEM((CH1, CHUNK), jnp.int32),
        pltpu.VMEM((CH1, CHUNK), jnp.int32),
        pltpu.VMEM((4, CHUNK, IN_F), jnp.float32),
        pltpu.VMEM((CHUNK, IN_F), jnp.float32),
        pltpu.VMEM_SHARED((NPAD, IN_F), jnp.float32),
        pltpu.SemaphoreType.DMA((4,)),
        pltpu.SemaphoreType.DMA((4,)),
    ],
)
def _pass1_kernel(src_hbm, dst_hbm, xs_hbm, acc_out,
                  sidx, didx, gbuf, zbuf, acc_sh, gsem, ssem):
    c = lax.axis_index("c")
    s = lax.axis_index("s")
    base = c * (ECH // NC) + s * CH1

    pltpu.sync_copy(src_hbm.at[pl.ds(base, CH1)], sidx)
    pltpu.sync_copy(dst_hbm.at[pl.ds(base, CH1)], didx)

    zeros = jnp.zeros((L,), jnp.float32)

    def zbody(i, _):
        zbuf[i, pl.ds(0, L)] = zeros
        return 0

    lax.fori_loop(0, CHUNK, zbody, 0)
    row0 = s * RPT
    for k in range(RFULL):
        pltpu.sync_copy(zbuf, acc_sh.at[pl.ds(row0 + k * CHUNK, CHUNK)])
    pltpu.sync_copy(zbuf.at[pl.ds(0, RREM)],
                    acc_sh.at[pl.ds(row0 + RFULL * CHUNK, RREM)])
    plsc.subcore_barrier()

    def gstart(j, b):
        pltpu.make_async_copy(
            xs_hbm.at[sidx.at[j]], gbuf.at[b], gsem.at[b]).start()

    def gwait(b):
        pltpu.make_async_copy(
            xs_hbm.at[sidx.at[0]], gbuf.at[b], gsem.at[b]).wait()

    def sstart(j, b):
        pltpu.make_async_copy(
            gbuf.at[b], acc_sh.at[didx.at[j]], ssem.at[b]).start(add=True)

    def swait(b):
        pltpu.make_async_copy(
            gbuf.at[b], acc_sh.at[didx.at[0]], ssem.at[b]).wait()

    _run_chunks(CH1, gstart, sstart, gwait, swait)
    plsc.subcore_barrier()

    for k in range(RFULL):
        pltpu.sync_copy(acc_sh.at[pl.ds(row0 + k * CHUNK, CHUNK)], zbuf)
        pltpu.sync_copy(zbuf, acc_out.at[c, pl.ds(row0 + k * CHUNK, CHUNK)])
    pltpu.sync_copy(acc_sh.at[pl.ds(row0 + RFULL * CHUNK, RREM)],
                    zbuf.at[pl.ds(0, RREM)])
    pltpu.sync_copy(zbuf.at[pl.ds(0, RREM)],
                    acc_out.at[c, pl.ds(row0 + RFULL * CHUNK, RREM)])


def _mid_body(acc1_ref, xs_ref, dinv_ref, w1_ref, b1_ref, hs_ref):
    a = acc1_ref[0] + acc1_ref[1] + xs_ref[...]
    dv = dinv_ref[0, 0, :]
    y1 = a * dv[:, None]
    h = jnp.dot(y1, w1_ref[...], preferred_element_type=jnp.float32,
                precision=lax.Precision.HIGHEST)
    h = jnp.maximum(h + b1_ref[...], 0.0)
    hs = h * dv[:, None]
    hs_ref[0] = hs[:, : H1 // 2]
    hs_ref[1] = hs[:, H1 // 2:]


def _mid(acc1, xs, dinv2d, W1, b1):
    return pl.pallas_call(
        _mid_body,
        grid=(NBLK,),
        in_specs=[
            pl.BlockSpec((NC, CHUNK, IN_F), lambda i: (0, i, 0)),
            pl.BlockSpec((CHUNK, IN_F), lambda i: (i, 0)),
            pl.BlockSpec((1, 1, CHUNK), lambda i: (i, 0, 0)),
            pl.BlockSpec((IN_F, H1), lambda i: (0, 0)),
            pl.BlockSpec((1, H1), lambda i: (0, 0)),
        ],
        out_specs=pl.BlockSpec((NC, CHUNK, H1 // 2), lambda i: (0, i, 0)),
        out_shape=jax.ShapeDtypeStruct((NC, NPAD, H1 // 2), jnp.float32),
        compiler_params=pltpu.CompilerParams(
            dimension_semantics=("arbitrary",)),
    )(acc1, xs, dinv2d, W1, b1)


@functools.partial(
    pl.kernel,
    out_type=jax.ShapeDtypeStruct((NC, NPAD, H1 // 2), jnp.float32),
    mesh=_mesh,
    compiler_params=_sc_params,
    scratch_types=[
        pltpu.VMEM((BLK2, CHUNK), jnp.int32),
        pltpu.VMEM((BLK2, CHUNK), jnp.int32),
        pltpu.VMEM((4, CHUNK, H1 // 2), jnp.float32),
        pltpu.VMEM((CHUNK, H1 // 2), jnp.float32),
        pltpu.VMEM_SHARED((NPAD, H1 // 2), jnp.float32),
        pltpu.SemaphoreType.DMA((4,)),
        pltpu.SemaphoreType.DMA((4,)),
    ],
)
def _pass2_kernel(src_hbm, dst_hbm, hs_hbm, acc_out,
                  sidx, didx, gbuf, zbuf, acc_sh, gsem, ssem):
    c = lax.axis_index("c")
    s = lax.axis_index("s")
    base = s * CH2

    zeros = jnp.zeros((L,), jnp.float32)

    def zbody(i, _):
        zbuf[i, pl.ds(0, L)] = zeros
        zbuf[i, pl.ds(L, L)] = zeros
        return 0

    lax.fori_loop(0, CHUNK, zbody, 0)
    row0 = s * RPT
    for k in range(RFULL):
        pltpu.sync_copy(zbuf, acc_sh.at[pl.ds(row0 + k * CHUNK, CHUNK)])
    pltpu.sync_copy(zbuf.at[pl.ds(0, RREM)],
                    acc_sh.at[pl.ds(row0 + RFULL * CHUNK, RREM)])
    plsc.subcore_barrier()

    def gstart(j, b):
        pltpu.make_async_copy(
            hs_hbm.at[c].at[sidx.at[j]], gbuf.at[b], gsem.at[b]).start()

    def gwait(b):
        pltpu.make_async_copy(
            hs_hbm.at[c].at[sidx.at[0]], gbuf.at[b], gsem.at[b]).wait()

    def sstart(j, b):
        pltpu.make_async_copy(
            gbuf.at[b], acc_sh.at[didx.at[j]], ssem.at[b]).start(add=True)

    def swait(b):
        pltpu.make_async_copy(
            gbuf.at[b], acc_sh.at[didx.at[0]], ssem.at[b]).wait()

    def blk_body(bb, _):
        pltpu.sync_copy(src_hbm.at[pl.ds(base + bb * BLK2, BLK2)], sidx)
        pltpu.sync_copy(dst_hbm.at[pl.ds(base + bb * BLK2, BLK2)], didx)
        _run_chunks(BLK2, gstart, sstart, gwait, swait)
        return 0

    lax.fori_loop(0, NB2, blk_body, 0)
    plsc.subcore_barrier()

    for k in range(RFULL):
        pltpu.sync_copy(acc_sh.at[pl.ds(row0 + k * CHUNK, CHUNK)], zbuf)
        pltpu.sync_copy(zbuf, acc_out.at[c, pl.ds(row0 + k * CHUNK, CHUNK)])
    pltpu.sync_copy(acc_sh.at[pl.ds(row0 + RFULL * CHUNK, RREM)],
                    zbuf.at[pl.ds(0, RREM)])
    pltpu.sync_copy(zbuf.at[pl.ds(0, RREM)],
                    acc_out.at[c, pl.ds(row0 + RFULL * CHUNK, RREM)])


def _out_body(acc2_ref, hs_ref, dinv_ref, w2_ref, wl_ref, b2_ref, bl_ref,
              out_ref):
    dv = dinv_ref[0, 0, :]
    y = jnp.concatenate(
        [acc2_ref[0] + hs_ref[0], acc2_ref[1] + hs_ref[1]], axis=1)
    y = y * dv[:, None]
    wf = jnp.dot(w2_ref[...], wl_ref[...],
                 preferred_element_type=jnp.float32,
                 precision=lax.Precision.HIGHEST)
    bf = jnp.dot(b2_ref[...], wl_ref[...],
                 preferred_element_type=jnp.float32,
                 precision=lax.Precision.HIGHEST) + bl_ref[...]
    out_ref[...] = jnp.dot(y, wf, preferred_element_type=jnp.float32,
                           precision=lax.Precision.HIGHEST) + bf


def _final(acc2, hs, dinv2d, W2, Wl, b2, bl):
    return pl.pallas_call(
        _out_body,
        grid=(NBLK,),
        in_specs=[
            pl.BlockSpec((NC, CHUNK, H1 // 2), lambda i: (0, i, 0)),
            pl.BlockSpec((NC, CHUNK, H1 // 2), lambda i: (0, i, 0)),
            pl.BlockSpec((1, 1, CHUNK), lambda i: (i, 0, 0)),
            pl.BlockSpec((H1, H2), lambda i: (0, 0)),
            pl.BlockSpec((H2, OUT_F), lambda i: (0, 0)),
            pl.BlockSpec((1, H2), lambda i: (0, 0)),
            pl.BlockSpec((1, OUT_F), lambda i: (0, 0)),
        ],
        out_specs=pl.BlockSpec((CHUNK, OUT_F), lambda i: (i, 0)),
        out_shape=jax.ShapeDtypeStruct((NPAD, OUT_F), jnp.float32),
        compiler_params=pltpu.CompilerParams(
            dimension_semantics=("arbitrary",)),
    )(acc2, hs, dinv2d, W2, Wl, b2, bl)


def kernel(node_matrix, graph, W1, b1, W2, b2, Wl, bl):
    E = graph.shape[1]
    src = graph[0]
    dst = graph[1]
    sent = N + jnp.arange(E_PAD - E, dtype=jnp.int32) % (NPAD - N)
    srcp = jnp.concatenate([src, sent]).reshape(ECH, CHUNK)
    dstp_flat = jnp.concatenate([dst, sent])
    dstp = dstp_flat.reshape(ECH, CHUNK)

    xpad = jnp.zeros((NPAD, IN_F), jnp.float32).at[:N].set(node_matrix)

    degp = _deg_kernel(dstp_flat)
    dinv2d, xs = _dinv_xs(degp, xpad)
    acc1 = _pass1_kernel(srcp, dstp, xs)
    hs = _mid(acc1, xs, dinv2d, W1, b1.reshape(1, H1))
    acc2 = _pass2_kernel(srcp, dstp, hs)
    outp = _final(acc2, hs, dinv2d, W2, Wl,
                  b2.reshape(1, H2), bl.reshape(1, OUT_F))
    return outp[:N]


# speedup vs baseline: 36.9896x; 1.9004x over previous
import functools

import jax
import jax.numpy as jnp
from jax import lax
from jax.experimental import pallas as pl
from jax.experimental.pallas import tpu as pltpu
from jax.experimental.pallas import tpu_sc as plsc

N = 50000
IN_F = 16
H1 = 64
H2 = 128
OUT_F = 16

NC = 2
NS = 16
L = 16
NW = NC * NS

CHUNK = 128
NPAD = 50048
NBLK = NPAD // CHUNK
TCB = 2176
TCG = NPAD // TCB
E_PAD = 819200
ECH = E_PAD // CHUNK
CH1 = ECH // NW
CH2 = ECH // NS
BLK2 = 40
NB2 = CH2 // BLK2
RPT = NPAD // NS
RFULL = RPT // CHUNK
RREM = RPT - RFULL * CHUNK

_mesh = plsc.VectorSubcoreMesh(core_axis_name="c", subcore_axis_name="s")
_sc_params = pltpu.CompilerParams(
    use_tc_tiling_on_sc=False, needs_layout_passes=False)


@functools.partial(
    pl.kernel,
    out_type=jax.ShapeDtypeStruct((NW, NPAD), jnp.float32),
    mesh=_mesh,
    compiler_params=_sc_params,
    scratch_types=[
        pltpu.VMEM((NPAD,), jnp.float32),
        pltpu.VMEM((E_PAD // NW,), jnp.int32),
    ],
)
def _deg_kernel(dst_hbm, degp_out, deg_v, idx_v):
    c = lax.axis_index("c")
    s = lax.axis_index("s")
    wid = s * NC + c
    epw = E_PAD // NW

    zeros = jnp.zeros((L,), jnp.float32)

    def zbody(i, _):
        deg_v[pl.ds(i * L, L)] = zeros
        return 0

    lax.fori_loop(0, NPAD // L, zbody, 0)

    pltpu.sync_copy(dst_hbm.at[pl.ds(wid * epw, epw)], idx_v)

    ones = jnp.full((L,), 1.0, jnp.float32)

    def body(i, _):
        iv = idx_v[pl.ds(i * L, L)]
        plsc.addupdate_scatter(deg_v, [iv], ones)
        return 0

    lax.fori_loop(0, epw // L, body, 0)
    pltpu.sync_copy(deg_v, degp_out.at[wid])


def _dinv_xs_body(degp_ref, x_ref, dinv_ref, xs_ref):
    deg = jnp.sum(degp_ref[...], axis=0) + 1.0
    dv = lax.rsqrt(deg)
    dinv_ref[...] = dv[:, None]
    xs_ref[...] = x_ref[...] * dv[:, None]


def _dinv_xs(degp, xpad):
    return pl.pallas_call(
        _dinv_xs_body,
        grid=(TCG,),
        in_specs=[
            pl.BlockSpec((NW, TCB), lambda i: (0, i)),
            pl.BlockSpec((TCB, IN_F), lambda i: (i, 0)),
        ],
        out_specs=[
            pl.BlockSpec((TCB, 1), lambda i: (i, 0)),
            pl.BlockSpec((TCB, IN_F), lambda i: (i, 0)),
        ],
        out_shape=[
            jax.ShapeDtypeStruct((NPAD, 1), jnp.float32),
            jax.ShapeDtypeStruct((NPAD, IN_F), jnp.float32),
        ],
        compiler_params=pltpu.CompilerParams(
            dimension_semantics=("arbitrary",)),
    )(degp, xpad)


def _run_chunks(nchunks, gstart, sstart, gwait, swait):
    kmax = nchunks // 4
    gstart(0, 0)
    gstart(1, 1)

    def body(kk, _):
        for b in range(4):
            j = 4 * kk + b
            gwait(b)
            sstart(j, b)
            tgt = (b + 2) % 4
            if b < 2:
                @pl.when(kk > 0)
                def _():
                    swait(tgt)

                gstart(j + 2, tgt)
            else:
                @pl.when(kk < kmax - 1)
                def _():
                    swait(tgt)
                    gstart(j + 2, tgt)
        return 0

    lax.fori_loop(0, kmax, body, 0)
    for b in range(4):
        swait(b)


@functools.partial(
    pl.kernel,
    out_type=jax.ShapeDtypeStruct((NC, NPAD, IN_F), jnp.float32),
    mesh=_mesh,
    compiler_params=_sc_params,
    scratch_types=[
        pltpu.VMEM((CH1, CHUNK), jnp.int32),
        pltpu.VMEM((CH1, CHUNK), jnp.int32),
        pltpu.VMEM((4, CHUNK, IN_F), jnp.float32),
        pltpu.VMEM((CHUNK, IN_F), jnp.float32),
        pltpu.VMEM_SHARED((NPAD, IN_F), jnp.float32),
        pltpu.SemaphoreType.DMA((4,)),
        pltpu.SemaphoreType.DMA((4,)),
    ],
)
def _pass1_kernel(src_hbm, dst_hbm, xs_hbm, acc_out,
                  sidx, didx, gbuf, zbuf, acc_sh, gsem, ssem):
    c = lax.axis_index("c")
    s = lax.axis_index("s")
    base = c * (ECH // NC) + s * CH1

    pltpu.sync_copy(src_hbm.at[pl.ds(base, CH1)], sidx)
    pltpu.sync_copy(dst_hbm.at[pl.ds(base, CH1)], didx)

    zeros = jnp.zeros((L,), jnp.float32)

    def zbody(i, _):
        zbuf[i, pl.ds(0, L)] = zeros
        return 0

    lax.fori_loop(0, CHUNK, zbody, 0)
    row0 = s * RPT
    for k in range(RFULL):
        pltpu.sync_copy(zbuf, acc_sh.at[pl.ds(row0 + k * CHUNK, CHUNK)])
    pltpu.sync_copy(zbuf.at[pl.ds(0, RREM)],
                    acc_sh.at[pl.ds(row0 + RFULL * CHUNK, RREM)])
    plsc.subcore_barrier()

    def gstart(j, b):
        pltpu.make_async_copy(
            xs_hbm.at[sidx.at[j]], gbuf.at[b], gsem.at[b]).start()

    def gwait(b):
        pltpu.make_async_copy(
            xs_hbm.at[sidx.at[0]], gbuf.at[b], gsem.at[b]).wait()

    def sstart(j, b):
        pltpu.make_async_copy(
            gbuf.at[b], acc_sh.at[didx.at[j]], ssem.at[b]).start(add=True)

    def swait(b):
        pltpu.make_async_copy(
            gbuf.at[b], acc_sh.at[didx.at[0]], ssem.at[b]).wait()

    _run_chunks(CH1, gstart, sstart, gwait, swait)
    plsc.subcore_barrier()

    for k in range(RFULL):
        pltpu.sync_copy(acc_sh.at[pl.ds(row0 + k * CHUNK, CHUNK)], zbuf)
        pltpu.sync_copy(zbuf, acc_out.at[c, pl.ds(row0 + k * CHUNK, CHUNK)])
    pltpu.sync_copy(acc_sh.at[pl.ds(row0 + RFULL * CHUNK, RREM)],
                    zbuf.at[pl.ds(0, RREM)])
    pltpu.sync_copy(zbuf.at[pl.ds(0, RREM)],
                    acc_out.at[c, pl.ds(row0 + RFULL * CHUNK, RREM)])


def _mid_body(acc1_ref, xs_ref, dinv_ref, w1_ref, b1_ref, hs_ref):
    a = acc1_ref[0] + acc1_ref[1] + xs_ref[...]
    dv = dinv_ref[...]
    y1 = a * dv
    h = jnp.dot(y1, w1_ref[...], preferred_element_type=jnp.float32,
                precision=lax.Precision.HIGHEST)
    h = jnp.maximum(h + b1_ref[...], 0.0)
    hs = h * dv
    hs_ref[0] = hs[:, : H1 // 2]
    hs_ref[1] = hs[:, H1 // 2:]


def _mid(acc1, xs, dinv2d, W1, b1):
    return pl.pallas_call(
        _mid_body,
        grid=(TCG,),
        in_specs=[
            pl.BlockSpec((NC, TCB, IN_F), lambda i: (0, i, 0)),
            pl.BlockSpec((TCB, IN_F), lambda i: (i, 0)),
            pl.BlockSpec((TCB, 1), lambda i: (i, 0)),
            pl.BlockSpec((IN_F, H1), lambda i: (0, 0)),
            pl.BlockSpec((1, H1), lambda i: (0, 0)),
        ],
        out_specs=pl.BlockSpec((NC, TCB, H1 // 2), lambda i: (0, i, 0)),
        out_shape=jax.ShapeDtypeStruct((NC, NPAD, H1 // 2), jnp.float32),
        compiler_params=pltpu.CompilerParams(
            dimension_semantics=("arbitrary",)),
    )(acc1, xs, dinv2d, W1, b1)


@functools.partial(
    pl.kernel,
    out_type=jax.ShapeDtypeStruct((NC, NPAD, H1 // 2), jnp.float32),
    mesh=_mesh,
    compiler_params=_sc_params,
    scratch_types=[
        pltpu.VMEM((BLK2, CHUNK), jnp.int32),
        pltpu.VMEM((BLK2, CHUNK), jnp.int32),
        pltpu.VMEM((4, CHUNK, H1 // 2), jnp.float32),
        pltpu.VMEM((CHUNK, H1 // 2), jnp.float32),
        pltpu.VMEM_SHARED((NPAD, H1 // 2), jnp.float32),
        pltpu.SemaphoreType.DMA((4,)),
        pltpu.SemaphoreType.DMA((4,)),
    ],
)
def _pass2_kernel(src_hbm, dst_hbm, hs_hbm, acc_out,
                  sidx, didx, gbuf, zbuf, acc_sh, gsem, ssem):
    c = lax.axis_index("c")
    s = lax.axis_index("s")
    base = s * CH2

    zeros = jnp.zeros((L,), jnp.float32)

    def zbody(i, _):
        zbuf[i, pl.ds(0, L)] = zeros
        zbuf[i, pl.ds(L, L)] = zeros
        return 0

    lax.fori_loop(0, CHUNK, zbody, 0)
    row0 = s * RPT
    for k in range(RFULL):
        pltpu.sync_copy(zbuf, acc_sh.at[pl.ds(row0 + k * CHUNK, CHUNK)])
    pltpu.sync_copy(zbuf.at[pl.ds(0, RREM)],
                    acc_sh.at[pl.ds(row0 + RFULL * CHUNK, RREM)])
    plsc.subcore_barrier()

    def gstart(j, b):
        pltpu.make_async_copy(
            hs_hbm.at[c].at[sidx.at[j]], gbuf.at[b], gsem.at[b]).start()

    def gwait(b):
        pltpu.make_async_copy(
            hs_hbm.at[c].at[sidx.at[0]], gbuf.at[b], gsem.at[b]).wait()

    def sstart(j, b):
        pltpu.make_async_copy(
            gbuf.at[b], acc_sh.at[didx.at[j]], ssem.at[b]).start(add=True)

    def swait(b):
        pltpu.make_async_copy(
            gbuf.at[b], acc_sh.at[didx.at[0]], ssem.at[b]).wait()

    def blk_body(bb, _):
        pltpu.sync_copy(src_hbm.at[pl.ds(base + bb * BLK2, BLK2)], sidx)
        pltpu.sync_copy(dst_hbm.at[pl.ds(base + bb * BLK2, BLK2)], didx)
        _run_chunks(BLK2, gstart, sstart, gwait, swait)
        return 0

    lax.fori_loop(0, NB2, blk_body, 0)
    plsc.subcore_barrier()

    for k in range(RFULL):
        pltpu.sync_copy(acc_sh.at[pl.ds(row0 + k * CHUNK, CHUNK)], zbuf)
        pltpu.sync_copy(zbuf, acc_out.at[c, pl.ds(row0 + k * CHUNK, CHUNK)])
    pltpu.sync_copy(acc_sh.at[pl.ds(row0 + RFULL * CHUNK, RREM)],
                    zbuf.at[pl.ds(0, RREM)])
    pltpu.sync_copy(zbuf.at[pl.ds(0, RREM)],
                    acc_out.at[c, pl.ds(row0 + RFULL * CHUNK, RREM)])


def _out_body(acc2_ref, hs_ref, dinv_ref, w2_ref, wl_ref, b2_ref, bl_ref,
              out_ref):
    dv = dinv_ref[...]
    y = jnp.concatenate(
        [acc2_ref[0] + hs_ref[0], acc2_ref[1] + hs_ref[1]], axis=1)
    y = y * dv
    wf = jnp.dot(w2_ref[...], wl_ref[...],
                 preferred_element_type=jnp.float32,
                 precision=lax.Precision.HIGHEST)
    bf = jnp.dot(b2_ref[...], wl_ref[...],
                 preferred_element_type=jnp.float32,
                 precision=lax.Precision.HIGHEST) + bl_ref[...]
    out_ref[...] = jnp.dot(y, wf, preferred_element_type=jnp.float32,
                           precision=lax.Precision.HIGHEST) + bf


def _final(acc2, hs, dinv2d, W2, Wl, b2, bl):
    return pl.pallas_call(
        _out_body,
        grid=(TCG,),
        in_specs=[
            pl.BlockSpec((NC, TCB, H1 // 2), lambda i: (0, i, 0)),
            pl.BlockSpec((NC, TCB, H1 // 2), lambda i: (0, i, 0)),
            pl.BlockSpec((TCB, 1), lambda i: (i, 0)),
            pl.BlockSpec((H1, H2), lambda i: (0, 0)),
            pl.BlockSpec((H2, OUT_F), lambda i: (0, 0)),
            pl.BlockSpec((1, H2), lambda i: (0, 0)),
            pl.BlockSpec((1, OUT_F), lambda i: (0, 0)),
        ],
        out_specs=pl.BlockSpec((TCB, OUT_F), lambda i: (i, 0)),
        out_shape=jax.ShapeDtypeStruct((NPAD, OUT_F), jnp.float32),
        compiler_params=pltpu.CompilerParams(
            dimension_semantics=("arbitrary",)),
    )(acc2, hs, dinv2d, W2, Wl, b2, bl)


def kernel(node_matrix, graph, W1, b1, W2, b2, Wl, bl):
    E = graph.shape[1]
    src = graph[0]
    dst = graph[1]
    sent = N + jnp.arange(E_PAD - E, dtype=jnp.int32) % (NPAD - N)
    srcp = jnp.concatenate([src, sent]).reshape(ECH, CHUNK)
    dstp_flat = jnp.concatenate([dst, sent])
    dstp = dstp_flat.reshape(ECH, CHUNK)

    xpad = jnp.zeros((NPAD, IN_F), jnp.float32).at[:N].set(node_matrix)

    degp = _deg_kernel(dstp_flat)
    dinv2d, xs = _dinv_xs(degp, xpad)
    acc1 = _pass1_kernel(srcp, dstp, xs)
    hs = _mid(acc1, xs, dinv2d, W1, b1.reshape(1, H1))
    acc2 = _pass2_kernel(srcp, dstp, hs)
    outp = _final(acc2, hs, dinv2d, W2, Wl,
                  b2.reshape(1, H2), bl.reshape(1, OUT_F))
    return outp[:N]


# speedup vs baseline: 41.9265x; 1.1335x over previous
import functools

import jax
import jax.numpy as jnp
from jax import lax
from jax.experimental import pallas as pl
from jax.experimental.pallas import tpu as pltpu
from jax.experimental.pallas import tpu_sc as plsc

N = 50000
IN_F = 16
H1 = 64
H2 = 128
OUT_F = 16

NC = 2
NS = 16
L = 16
NW = NC * NS

CHUNK = 128
NPAD = 50048
NBLK = NPAD // CHUNK
TCB = 2176
TCG = NPAD // TCB
E_PAD = 819200
ECH = E_PAD // CHUNK
CH1 = ECH // NW
CH2 = ECH // NS
BLK2 = 40
NB2 = CH2 // BLK2
RPT = NPAD // NS
RFULL = RPT // CHUNK
RREM = RPT - RFULL * CHUNK

_mesh = plsc.VectorSubcoreMesh(core_axis_name="c", subcore_axis_name="s")
_sc_params = pltpu.CompilerParams(
    use_tc_tiling_on_sc=False, needs_layout_passes=False)


@functools.partial(
    pl.kernel,
    out_type=jax.ShapeDtypeStruct((NW, NPAD), jnp.float32),
    mesh=_mesh,
    compiler_params=_sc_params,
    scratch_types=[
        pltpu.VMEM((NPAD,), jnp.float32),
        pltpu.VMEM((E_PAD // NW,), jnp.int32),
    ],
)
def _deg_kernel(dst_hbm, degp_out, deg_v, idx_v):
    c = lax.axis_index("c")
    s = lax.axis_index("s")
    wid = s * NC + c
    epw = E_PAD // NW

    zeros = jnp.zeros((L,), jnp.float32)

    def zbody(i, _):
        deg_v[pl.ds(i * L, L)] = zeros
        return 0

    lax.fori_loop(0, NPAD // L, zbody, 0)

    pltpu.sync_copy(dst_hbm.at[pl.ds(wid * epw, epw)], idx_v)

    ones = jnp.full((L,), 1.0, jnp.float32)

    def body(i, _):
        iv = idx_v[pl.ds(i * L, L)]
        plsc.addupdate_scatter(deg_v, [iv], ones)
        return 0

    lax.fori_loop(0, epw // L, body, 0)
    pltpu.sync_copy(deg_v, degp_out.at[wid])


def _dinv_xs_body(degp_ref, x_ref, dinv_ref, xs_ref):
    deg = jnp.sum(degp_ref[...], axis=0) + 1.0
    dv = lax.rsqrt(deg)
    dinv_ref[...] = dv[:, None]
    xs_ref[...] = x_ref[...] * dv[:, None]


def _dinv_xs(degp, xpad):
    return pl.pallas_call(
        _dinv_xs_body,
        grid=(TCG,),
        in_specs=[
            pl.BlockSpec((NW, TCB), lambda i: (0, i)),
            pl.BlockSpec((TCB, IN_F), lambda i: (i, 0)),
        ],
        out_specs=[
            pl.BlockSpec((TCB, 1), lambda i: (i, 0)),
            pl.BlockSpec((TCB, IN_F), lambda i: (i, 0)),
        ],
        out_shape=[
            jax.ShapeDtypeStruct((NPAD, 1), jnp.float32),
            jax.ShapeDtypeStruct((NPAD, IN_F), jnp.float32),
        ],
        compiler_params=pltpu.CompilerParams(
            dimension_semantics=("arbitrary",)),
    )(degp, xpad)


def _run_chunks(nchunks, gstart, sstart, gwait, swait):
    kmax = nchunks // 4
    gstart(0, 0)
    gstart(1, 1)
    gstart(2, 2)

    def body(kk, _):
        for b in range(4):
            j = 4 * kk + b
            gwait(b)
            sstart(j, b)
            tgt = (b + 3) % 4
            if b == 0:
                @pl.when(kk > 0)
                def _():
                    swait(tgt)

                gstart(j + 3, tgt)
            else:
                swait(tgt)

                @pl.when(kk < kmax - 1)
                def _():
                    gstart(j + 3, tgt)
        return 0

    lax.fori_loop(0, kmax, body, 0)
    swait(3)


@functools.partial(
    pl.kernel,
    out_type=jax.ShapeDtypeStruct((NC, NPAD, IN_F), jnp.float32),
    mesh=_mesh,
    compiler_params=_sc_params,
    scratch_types=[
        pltpu.VMEM((CH1, CHUNK), jnp.int32),
        pltpu.VMEM((CH1, CHUNK), jnp.int32),
        pltpu.VMEM((4, CHUNK, IN_F), jnp.float32),
        pltpu.VMEM((CHUNK, IN_F), jnp.float32),
        pltpu.VMEM_SHARED((NPAD, IN_F), jnp.float32),
        pltpu.SemaphoreType.DMA((4,)),
        pltpu.SemaphoreType.DMA((4,)),
    ],
)
def _pass1_kernel(src_hbm, dst_hbm, xs_hbm, acc_out,
                  sidx, didx, gbuf, zbuf, acc_sh, gsem, ssem):
    c = lax.axis_index("c")
    s = lax.axis_index("s")
    base = c * (ECH // NC) + s * CH1

    pltpu.sync_copy(src_hbm.at[pl.ds(base, CH1)], sidx)
    pltpu.sync_copy(dst_hbm.at[pl.ds(base, CH1)], didx)

    zeros = jnp.zeros((L,), jnp.float32)

    def zbody(i, _):
        zbuf[i, pl.ds(0, L)] = zeros
        return 0

    lax.fori_loop(0, CHUNK, zbody, 0)
    row0 = s * RPT
    for k in range(RFULL):
        pltpu.sync_copy(zbuf, acc_sh.at[pl.ds(row0 + k * CHUNK, CHUNK)])
    pltpu.sync_copy(zbuf.at[pl.ds(0, RREM)],
                    acc_sh.at[pl.ds(row0 + RFULL * CHUNK, RREM)])
    plsc.subcore_barrier()

    def gstart(j, b):
        pltpu.make_async_copy(
            xs_hbm.at[sidx.at[j]], gbuf.at[b], gsem.at[b]).start()

    def gwait(b):
        pltpu.make_async_copy(
            xs_hbm.at[sidx.at[0]], gbuf.at[b], gsem.at[b]).wait()

    def sstart(j, b):
        pltpu.make_async_copy(
            gbuf.at[b], acc_sh.at[didx.at[j]], ssem.at[b]).start(add=True)

    def swait(b):
        pltpu.make_async_copy(
            gbuf.at[b], acc_sh.at[didx.at[0]], ssem.at[b]).wait()

    _run_chunks(CH1, gstart, sstart, gwait, swait)
    plsc.subcore_barrier()

    for k in range(RFULL):
        pltpu.sync_copy(acc_sh.at[pl.ds(row0 + k * CHUNK, CHUNK)], zbuf)
        pltpu.sync_copy(zbuf, acc_out.at[c, pl.ds(row0 + k * CHUNK, CHUNK)])
    pltpu.sync_copy(acc_sh.at[pl.ds(row0 + RFULL * CHUNK, RREM)],
                    zbuf.at[pl.ds(0, RREM)])
    pltpu.sync_copy(zbuf.at[pl.ds(0, RREM)],
                    acc_out.at[c, pl.ds(row0 + RFULL * CHUNK, RREM)])


def _mid_body(acc1_ref, xs_ref, dinv_ref, w1_ref, b1_ref, hs_ref):
    a = acc1_ref[0] + acc1_ref[1] + xs_ref[...]
    dv = dinv_ref[...]
    y1 = a * dv
    h = jnp.dot(y1, w1_ref[...], preferred_element_type=jnp.float32,
                precision=lax.Precision.HIGHEST)
    h = jnp.maximum(h + b1_ref[...], 0.0)
    hs = h * dv
    hs_ref[0] = hs[:, : H1 // 2]
    hs_ref[1] = hs[:, H1 // 2:]


def _mid(acc1, xs, dinv2d, W1, b1):
    return pl.pallas_call(
        _mid_body,
        grid=(TCG,),
        in_specs=[
            pl.BlockSpec((NC, TCB, IN_F), lambda i: (0, i, 0)),
            pl.BlockSpec((TCB, IN_F), lambda i: (i, 0)),
            pl.BlockSpec((TCB, 1), lambda i: (i, 0)),
            pl.BlockSpec((IN_F, H1), lambda i: (0, 0)),
            pl.BlockSpec((1, H1), lambda i: (0, 0)),
        ],
        out_specs=pl.BlockSpec((NC, TCB, H1 // 2), lambda i: (0, i, 0)),
        out_shape=jax.ShapeDtypeStruct((NC, NPAD, H1 // 2), jnp.float32),
        compiler_params=pltpu.CompilerParams(
            dimension_semantics=("arbitrary",)),
    )(acc1, xs, dinv2d, W1, b1)


@functools.partial(
    pl.kernel,
    out_type=jax.ShapeDtypeStruct((NC, NPAD, H1 // 2), jnp.float32),
    mesh=_mesh,
    compiler_params=_sc_params,
    scratch_types=[
        pltpu.VMEM((BLK2, CHUNK), jnp.int32),
        pltpu.VMEM((BLK2, CHUNK), jnp.int32),
        pltpu.VMEM((4, CHUNK, H1 // 2), jnp.float32),
        pltpu.VMEM((CHUNK, H1 // 2), jnp.float32),
        pltpu.VMEM_SHARED((NPAD, H1 // 2), jnp.float32),
        pltpu.SemaphoreType.DMA((4,)),
        pltpu.SemaphoreType.DMA((4,)),
    ],
)
def _pass2_kernel(src_hbm, dst_hbm, hs_hbm, acc_out,
                  sidx, didx, gbuf, zbuf, acc_sh, gsem, ssem):
    c = lax.axis_index("c")
    s = lax.axis_index("s")
    base = s * CH2

    zeros = jnp.zeros((L,), jnp.float32)

    def zbody(i, _):
        zbuf[i, pl.ds(0, L)] = zeros
        zbuf[i, pl.ds(L, L)] = zeros
        return 0

    lax.fori_loop(0, CHUNK, zbody, 0)
    row0 = s * RPT
    for k in range(RFULL):
        pltpu.sync_copy(zbuf, acc_sh.at[pl.ds(row0 + k * CHUNK, CHUNK)])
    pltpu.sync_copy(zbuf.at[pl.ds(0, RREM)],
                    acc_sh.at[pl.ds(row0 + RFULL * CHUNK, RREM)])
    plsc.subcore_barrier()

    def gstart(j, b):
        pltpu.make_async_copy(
            hs_hbm.at[c].at[sidx.at[j]], gbuf.at[b], gsem.at[b]).start()

    def gwait(b):
        pltpu.make_async_copy(
            hs_hbm.at[c].at[sidx.at[0]], gbuf.at[b], gsem.at[b]).wait()

    def sstart(j, b):
        pltpu.make_async_copy(
            gbuf.at[b], acc_sh.at[didx.at[j]], ssem.at[b]).start(add=True)

    def swait(b):
        pltpu.make_async_copy(
            gbuf.at[b], acc_sh.at[didx.at[0]], ssem.at[b]).wait()

    def blk_body(bb, _):
        pltpu.sync_copy(src_hbm.at[pl.ds(base + bb * BLK2, BLK2)], sidx)
        pltpu.sync_copy(dst_hbm.at[pl.ds(base + bb * BLK2, BLK2)], didx)
        _run_chunks(BLK2, gstart, sstart, gwait, swait)
        return 0

    lax.fori_loop(0, NB2, blk_body, 0)
    plsc.subcore_barrier()

    for k in range(RFULL):
        pltpu.sync_copy(acc_sh.at[pl.ds(row0 + k * CHUNK, CHUNK)], zbuf)
        pltpu.sync_copy(zbuf, acc_out.at[c, pl.ds(row0 + k * CHUNK, CHUNK)])
    pltpu.sync_copy(acc_sh.at[pl.ds(row0 + RFULL * CHUNK, RREM)],
                    zbuf.at[pl.ds(0, RREM)])
    pltpu.sync_copy(zbuf.at[pl.ds(0, RREM)],
                    acc_out.at[c, pl.ds(row0 + RFULL * CHUNK, RREM)])


def _out_body(acc2_ref, hs_ref, dinv_ref, w2_ref, wl_ref, b2_ref, bl_ref,
              out_ref):
    dv = dinv_ref[...]
    y = jnp.concatenate(
        [acc2_ref[0] + hs_ref[0], acc2_ref[1] + hs_ref[1]], axis=1)
    y = y * dv
    wf = jnp.dot(w2_ref[...], wl_ref[...],
                 preferred_element_type=jnp.float32,
                 precision=lax.Precision.HIGHEST)
    bf = jnp.dot(b2_ref[...], wl_ref[...],
                 preferred_element_type=jnp.float32,
                 precision=lax.Precision.HIGHEST) + bl_ref[...]
    out_ref[...] = jnp.dot(y, wf, preferred_element_type=jnp.float32,
                           precision=lax.Precision.HIGHEST) + bf


def _final(acc2, hs, dinv2d, W2, Wl, b2, bl):
    return pl.pallas_call(
        _out_body,
        grid=(TCG,),
        in_specs=[
            pl.BlockSpec((NC, TCB, H1 // 2), lambda i: (0, i, 0)),
            pl.BlockSpec((NC, TCB, H1 // 2), lambda i: (0, i, 0)),
            pl.BlockSpec((TCB, 1), lambda i: (i, 0)),
            pl.BlockSpec((H1, H2), lambda i: (0, 0)),
            pl.BlockSpec((H2, OUT_F), lambda i: (0, 0)),
            pl.BlockSpec((1, H2), lambda i: (0, 0)),
            pl.BlockSpec((1, OUT_F), lambda i: (0, 0)),
        ],
        out_specs=pl.BlockSpec((TCB, OUT_F), lambda i: (i, 0)),
        out_shape=jax.ShapeDtypeStruct((N, OUT_F), jnp.float32),
        compiler_params=pltpu.CompilerParams(
            dimension_semantics=("arbitrary",)),
    )(acc2, hs, dinv2d, W2, Wl, b2, bl)


def kernel(node_matrix, graph, W1, b1, W2, b2, Wl, bl):
    E = graph.shape[1]
    src = graph[0]
    dst = graph[1]
    sent = N + jnp.arange(E_PAD - E, dtype=jnp.int32) % (NPAD - N)
    srcp = jnp.concatenate([src, sent]).reshape(ECH, CHUNK)
    dstp_flat = jnp.concatenate([dst, sent])
    dstp = dstp_flat.reshape(ECH, CHUNK)

    xpad = jnp.zeros((NPAD, IN_F), jnp.float32).at[:N].set(node_matrix)

    degp = _deg_kernel(dstp_flat)
    dinv2d, xs = _dinv_xs(degp, xpad)
    acc1 = _pass1_kernel(srcp, dstp, xs)
    hs = _mid(acc1, xs, dinv2d, W1, b1.reshape(1, H1))
    acc2 = _pass2_kernel(srcp, dstp, hs)
    outp = _final(acc2, hs, dinv2d, W2, Wl,
                  b2.reshape(1, H2), bl.reshape(1, OUT_F))
    return outp


# speedup vs baseline: 42.0129x; 1.0021x over previous
import functools

import jax
import jax.numpy as jnp
from jax import lax
from jax.experimental import pallas as pl
from jax.experimental.pallas import tpu as pltpu
from jax.experimental.pallas import tpu_sc as plsc

N = 50000
IN_F = 16
H1 = 64
H2 = 128
OUT_F = 16

NC = 2
NS = 16
L = 16
NW = NC * NS

CHUNK = 128
NPAD = 50048
NBLK = NPAD // CHUNK
TCB = 2176
TCG = NPAD // TCB
E_PAD = 819200
ECH = E_PAD // CHUNK
CH1 = ECH // NW
CH2 = ECH // NS
BLK2 = 40
NB2 = CH2 // BLK2
RPT = NPAD // NS
RFULL = RPT // CHUNK
RREM = RPT - RFULL * CHUNK

_mesh = plsc.VectorSubcoreMesh(core_axis_name="c", subcore_axis_name="s")
_sc_params = pltpu.CompilerParams(
    use_tc_tiling_on_sc=False, needs_layout_passes=False)


@functools.partial(
    pl.kernel,
    out_type=jax.ShapeDtypeStruct((NW, NPAD), jnp.float32),
    mesh=_mesh,
    compiler_params=_sc_params,
    scratch_types=[
        pltpu.VMEM((NPAD,), jnp.float32),
        pltpu.VMEM((CH1, CHUNK), jnp.int32),
    ],
)
def _deg_kernel(dst_hbm, degp_out, deg_v, idx_v):
    c = lax.axis_index("c")
    s = lax.axis_index("s")
    wid = s * NC + c
    epw = E_PAD // NW

    zeros = jnp.zeros((L,), jnp.float32)

    def zbody(i, _):
        deg_v[pl.ds(i * L, L)] = zeros
        return 0

    lax.fori_loop(0, NPAD // L, zbody, 0)

    pltpu.sync_copy(dst_hbm.at[pl.ds(wid * CH1, CH1)], idx_v)

    ones = jnp.full((L,), 1.0, jnp.float32)

    def body(i, _):
        r = i // (CHUNK // L)
        k = i % (CHUNK // L)
        iv = idx_v[r, pl.ds(k * L, L)]
        plsc.addupdate_scatter(deg_v, [iv], ones)
        return 0

    lax.fori_loop(0, epw // L, body, 0)
    pltpu.sync_copy(deg_v, degp_out.at[wid])


def _dinv_xs_body(degp_ref, x_ref, dinv_ref, xs_ref):
    deg = jnp.sum(degp_ref[...], axis=0) + 1.0
    dv = lax.rsqrt(deg)
    dinv_ref[...] = dv[:, None]
    xs_ref[...] = x_ref[...] * dv[:, None]


def _dinv_xs(degp, xpad):
    return pl.pallas_call(
        _dinv_xs_body,
        grid=(TCG,),
        in_specs=[
            pl.BlockSpec((NW, TCB), lambda i: (0, i)),
            pl.BlockSpec((TCB, IN_F), lambda i: (i, 0)),
        ],
        out_specs=[
            pl.BlockSpec((TCB, 1), lambda i: (i, 0)),
            pl.BlockSpec((TCB, IN_F), lambda i: (i, 0)),
        ],
        out_shape=[
            jax.ShapeDtypeStruct((NPAD, 1), jnp.float32),
            jax.ShapeDtypeStruct((NPAD, IN_F), jnp.float32),
        ],
        compiler_params=pltpu.CompilerParams(
            dimension_semantics=("arbitrary",)),
    )(degp, xpad)


def _run_chunks(nchunks, gstart, sstart, gwait, swait):
    kmax = nchunks // 4
    gstart(0, 0)
    gstart(1, 1)
    gstart(2, 2)

    def body(kk, _):
        for b in range(4):
            j = 4 * kk + b
            gwait(b)
            sstart(j, b)
            tgt = (b + 3) % 4
            if b == 0:
                @pl.when(kk > 0)
                def _():
                    swait(tgt)

                gstart(j + 3, tgt)
            else:
                swait(tgt)

                @pl.when(kk < kmax - 1)
                def _():
                    gstart(j + 3, tgt)
        return 0

    lax.fori_loop(0, kmax, body, 0)
    swait(3)


@functools.partial(
    pl.kernel,
    out_type=jax.ShapeDtypeStruct((NC, NPAD, IN_F), jnp.float32),
    mesh=_mesh,
    compiler_params=_sc_params,
    scratch_types=[
        pltpu.VMEM((CH1, CHUNK), jnp.int32),
        pltpu.VMEM((CH1, CHUNK), jnp.int32),
        pltpu.VMEM((4, CHUNK, IN_F), jnp.float32),
        pltpu.VMEM((CHUNK, IN_F), jnp.float32),
        pltpu.VMEM_SHARED((NPAD, IN_F), jnp.float32),
        pltpu.SemaphoreType.DMA((4,)),
        pltpu.SemaphoreType.DMA((4,)),
    ],
)
def _pass1_kernel(src_hbm, dst_hbm, xs_hbm, acc_out,
                  sidx, didx, gbuf, zbuf, acc_sh, gsem, ssem):
    c = lax.axis_index("c")
    s = lax.axis_index("s")
    base = c * (ECH // NC) + s * CH1

    pltpu.sync_copy(src_hbm.at[pl.ds(base, CH1)], sidx)
    pltpu.sync_copy(dst_hbm.at[pl.ds(base, CH1)], didx)

    zeros = jnp.zeros((L,), jnp.float32)

    def zbody(i, _):
        zbuf[i, pl.ds(0, L)] = zeros
        return 0

    lax.fori_loop(0, CHUNK, zbody, 0)
    row0 = s * RPT
    for k in range(RFULL):
        pltpu.sync_copy(zbuf, acc_sh.at[pl.ds(row0 + k * CHUNK, CHUNK)])
    pltpu.sync_copy(zbuf.at[pl.ds(0, RREM)],
                    acc_sh.at[pl.ds(row0 + RFULL * CHUNK, RREM)])
    plsc.subcore_barrier()

    def gstart(j, b):
        pltpu.make_async_copy(
            xs_hbm.at[sidx.at[j]], gbuf.at[b], gsem.at[b]).start()

    def gwait(b):
        pltpu.make_async_copy(
            xs_hbm.at[sidx.at[0]], gbuf.at[b], gsem.at[b]).wait()

    def sstart(j, b):
        pltpu.make_async_copy(
            gbuf.at[b], acc_sh.at[didx.at[j]], ssem.at[b]).start(add=True)

    def swait(b):
        pltpu.make_async_copy(
            gbuf.at[b], acc_sh.at[didx.at[0]], ssem.at[b]).wait()

    _run_chunks(CH1, gstart, sstart, gwait, swait)
    plsc.subcore_barrier()

    for k in range(RFULL):
        pltpu.sync_copy(acc_sh.at[pl.ds(row0 + k * CHUNK, CHUNK)], zbuf)
        pltpu.sync_copy(zbuf, acc_out.at[c, pl.ds(row0 + k * CHUNK, CHUNK)])
    pltpu.sync_copy(acc_sh.at[pl.ds(row0 + RFULL * CHUNK, RREM)],
                    zbuf.at[pl.ds(0, RREM)])
    pltpu.sync_copy(zbuf.at[pl.ds(0, RREM)],
                    acc_out.at[c, pl.ds(row0 + RFULL * CHUNK, RREM)])


def _mid_body(acc1_ref, xs_ref, dinv_ref, w1_ref, b1_ref, hs_ref):
    a = acc1_ref[0] + acc1_ref[1] + xs_ref[...]
    dv = dinv_ref[...]
    y1 = a * dv
    h = jnp.dot(y1, w1_ref[...], preferred_element_type=jnp.float32,
                precision=lax.Precision.HIGHEST)
    h = jnp.maximum(h + b1_ref[...], 0.0)
    hs = h * dv
    hs_ref[0] = hs[:, : H1 // 2]
    hs_ref[1] = hs[:, H1 // 2:]


def _mid(acc1, xs, dinv2d, W1, b1):
    return pl.pallas_call(
        _mid_body,
        grid=(TCG,),
        in_specs=[
            pl.BlockSpec((NC, TCB, IN_F), lambda i: (0, i, 0)),
            pl.BlockSpec((TCB, IN_F), lambda i: (i, 0)),
            pl.BlockSpec((TCB, 1), lambda i: (i, 0)),
            pl.BlockSpec((IN_F, H1), lambda i: (0, 0)),
            pl.BlockSpec((1, H1), lambda i: (0, 0)),
        ],
        out_specs=pl.BlockSpec((NC, TCB, H1 // 2), lambda i: (0, i, 0)),
        out_shape=jax.ShapeDtypeStruct((NC, NPAD, H1 // 2), jnp.float32),
        compiler_params=pltpu.CompilerParams(
            dimension_semantics=("arbitrary",)),
    )(acc1, xs, dinv2d, W1, b1)


@functools.partial(
    pl.kernel,
    out_type=jax.ShapeDtypeStruct((NC, NPAD, H1 // 2), jnp.float32),
    mesh=_mesh,
    compiler_params=_sc_params,
    scratch_types=[
        pltpu.VMEM((BLK2, CHUNK), jnp.int32),
        pltpu.VMEM((BLK2, CHUNK), jnp.int32),
        pltpu.VMEM((4, CHUNK, H1 // 2), jnp.float32),
        pltpu.VMEM((CHUNK, H1 // 2), jnp.float32),
        pltpu.VMEM_SHARED((NPAD, H1 // 2), jnp.float32),
        pltpu.SemaphoreType.DMA((4,)),
        pltpu.SemaphoreType.DMA((4,)),
    ],
)
def _pass2_kernel(src_hbm, dst_hbm, hs_hbm, acc_out,
                  sidx, didx, gbuf, zbuf, acc_sh, gsem, ssem):
    c = lax.axis_index("c")
    s = lax.axis_index("s")
    base = s * CH2

    zeros = jnp.zeros((L,), jnp.float32)

    def zbody(i, _):
        zbuf[i, pl.ds(0, L)] = zeros
        zbuf[i, pl.ds(L, L)] = zeros
        return 0

    lax.fori_loop(0, CHUNK, zbody, 0)
    row0 = s * RPT
    for k in range(RFULL):
        pltpu.sync_copy(zbuf, acc_sh.at[pl.ds(row0 + k * CHUNK, CHUNK)])
    pltpu.sync_copy(zbuf.at[pl.ds(0, RREM)],
                    acc_sh.at[pl.ds(row0 + RFULL * CHUNK, RREM)])
    plsc.subcore_barrier()

    def gstart(j, b):
        pltpu.make_async_copy(
            hs_hbm.at[c].at[sidx.at[j]], gbuf.at[b], gsem.at[b]).start()

    def gwait(b):
        pltpu.make_async_copy(
            hs_hbm.at[c].at[sidx.at[0]], gbuf.at[b], gsem.at[b]).wait()

    def sstart(j, b):
        pltpu.make_async_copy(
            gbuf.at[b], acc_sh.at[didx.at[j]], ssem.at[b]).start(add=True)

    def swait(b):
        pltpu.make_async_copy(
            gbuf.at[b], acc_sh.at[didx.at[0]], ssem.at[b]).wait()

    def blk_body(bb, _):
        pltpu.sync_copy(src_hbm.at[pl.ds(base + bb * BLK2, BLK2)], sidx)
        pltpu.sync_copy(dst_hbm.at[pl.ds(base + bb * BLK2, BLK2)], didx)
        _run_chunks(BLK2, gstart, sstart, gwait, swait)
        return 0

    lax.fori_loop(0, NB2, blk_body, 0)
    plsc.subcore_barrier()

    for k in range(RFULL):
        pltpu.sync_copy(acc_sh.at[pl.ds(row0 + k * CHUNK, CHUNK)], zbuf)
        pltpu.sync_copy(zbuf, acc_out.at[c, pl.ds(row0 + k * CHUNK, CHUNK)])
    pltpu.sync_copy(acc_sh.at[pl.ds(row0 + RFULL * CHUNK, RREM)],
                    zbuf.at[pl.ds(0, RREM)])
    pltpu.sync_copy(zbuf.at[pl.ds(0, RREM)],
                    acc_out.at[c, pl.ds(row0 + RFULL * CHUNK, RREM)])


def _out_body(acc2_ref, hs_ref, dinv_ref, w2_ref, wl_ref, b2_ref, bl_ref,
              out_ref):
    dv = dinv_ref[...]
    y = jnp.concatenate(
        [acc2_ref[0] + hs_ref[0], acc2_ref[1] + hs_ref[1]], axis=1)
    y = y * dv
    wf = jnp.dot(w2_ref[...], wl_ref[...],
                 preferred_element_type=jnp.float32,
                 precision=lax.Precision.HIGHEST)
    bf = jnp.dot(b2_ref[...], wl_ref[...],
                 preferred_element_type=jnp.float32,
                 precision=lax.Precision.HIGHEST) + bl_ref[...]
    out_ref[...] = jnp.dot(y, wf, preferred_element_type=jnp.float32,
                           precision=lax.Precision.HIGHEST) + bf


def _final(acc2, hs, dinv2d, W2, Wl, b2, bl):
    return pl.pallas_call(
        _out_body,
        grid=(TCG,),
        in_specs=[
            pl.BlockSpec((NC, TCB, H1 // 2), lambda i: (0, i, 0)),
            pl.BlockSpec((NC, TCB, H1 // 2), lambda i: (0, i, 0)),
            pl.BlockSpec((TCB, 1), lambda i: (i, 0)),
            pl.BlockSpec((H1, H2), lambda i: (0, 0)),
            pl.BlockSpec((H2, OUT_F), lambda i: (0, 0)),
            pl.BlockSpec((1, H2), lambda i: (0, 0)),
            pl.BlockSpec((1, OUT_F), lambda i: (0, 0)),
        ],
        out_specs=pl.BlockSpec((TCB, OUT_F), lambda i: (i, 0)),
        out_shape=jax.ShapeDtypeStruct((N, OUT_F), jnp.float32),
        compiler_params=pltpu.CompilerParams(
            dimension_semantics=("arbitrary",)),
    )(acc2, hs, dinv2d, W2, Wl, b2, bl)


def kernel(node_matrix, graph, W1, b1, W2, b2, Wl, bl):
    E = graph.shape[1]
    src = graph[0]
    dst = graph[1]
    sent = N + jnp.arange(E_PAD - E, dtype=jnp.int32) % (NPAD - N)
    dstp = jnp.concatenate([dst, sent]).reshape(ECH, CHUNK)
    srcp = jnp.concatenate([src, sent]).reshape(ECH, CHUNK)

    xpad = jnp.zeros((NPAD, IN_F), jnp.float32).at[:N].set(node_matrix)

    degp = _deg_kernel(dstp)
    dinv2d, xs = _dinv_xs(degp, xpad)
    acc1 = _pass1_kernel(srcp, dstp, xs)
    hs = _mid(acc1, xs, dinv2d, W1, b1.reshape(1, H1))
    acc2 = _pass2_kernel(srcp, dstp, hs)
    outp = _final(acc2, hs, dinv2d, W2, Wl,
                  b2.reshape(1, H2), bl.reshape(1, OUT_F))
    return outp


# speedup vs baseline: 42.6430x; 1.0150x over previous
import functools

import jax
import jax.numpy as jnp
from jax import lax
from jax.experimental import pallas as pl
from jax.experimental.pallas import tpu as pltpu
from jax.experimental.pallas import tpu_sc as plsc

N = 50000
IN_F = 16
H1 = 64
H2 = 128
OUT_F = 16

NC = 2
NS = 16
L = 16
NW = NC * NS

CHUNK = 128
NPAD = 50048
NBLK = NPAD // CHUNK
TCB = 2176
TCG = NPAD // TCB
E_PAD = 819200
ECH = E_PAD // CHUNK
CH1 = ECH // NW
CH2 = ECH // NS
BLK2 = 40
NB2 = CH2 // BLK2
RPT = NPAD // NS
RFULL = RPT // CHUNK
RREM = RPT - RFULL * CHUNK

_mesh = plsc.VectorSubcoreMesh(core_axis_name="c", subcore_axis_name="s")
_sc_params = pltpu.CompilerParams(
    use_tc_tiling_on_sc=False, needs_layout_passes=False)


@functools.partial(
    pl.kernel,
    out_type=jax.ShapeDtypeStruct((NW, NPAD), jnp.float32),
    mesh=_mesh,
    compiler_params=_sc_params,
    scratch_types=[
        pltpu.VMEM((NPAD,), jnp.float32),
        pltpu.VMEM((CH1, CHUNK), jnp.int32),
    ],
)
def _deg_kernel(dst_hbm, degp_out, deg_v, idx_v):
    c = lax.axis_index("c")
    s = lax.axis_index("s")
    wid = s * NC + c
    epw = E_PAD // NW

    zeros = jnp.zeros((L,), jnp.float32)

    def zbody(i, _):
        deg_v[pl.ds(i * L, L)] = zeros
        return 0

    lax.fori_loop(0, NPAD // L, zbody, 0)

    pltpu.sync_copy(dst_hbm.at[pl.ds(wid * CH1, CH1)], idx_v)

    ones = jnp.full((L,), 1.0, jnp.float32)

    def body(i, _):
        r = i // (CHUNK // L)
        k = i % (CHUNK // L)
        iv = idx_v[r, pl.ds(k * L, L)]
        plsc.addupdate_scatter(deg_v, [iv], ones)
        return 0

    lax.fori_loop(0, epw // L, body, 0)
    pltpu.sync_copy(deg_v, degp_out.at[wid])


def _dinv_xs_body(degp_ref, x_ref, dinv_ref, xs_ref):
    deg = jnp.sum(degp_ref[...], axis=0) + 1.0
    dv = lax.rsqrt(deg)
    dinv_ref[...] = dv[:, None]
    xs_ref[...] = x_ref[...] * dv[:, None]


def _dinv_xs(degp, xpad):
    return pl.pallas_call(
        _dinv_xs_body,
        grid=(TCG,),
        in_specs=[
            pl.BlockSpec((NW, TCB), lambda i: (0, i)),
            pl.BlockSpec((TCB, IN_F), lambda i: (i, 0)),
        ],
        out_specs=[
            pl.BlockSpec((TCB, 1), lambda i: (i, 0)),
            pl.BlockSpec((TCB, IN_F), lambda i: (i, 0)),
        ],
        out_shape=[
            jax.ShapeDtypeStruct((NPAD, 1), jnp.float32),
            jax.ShapeDtypeStruct((NPAD, IN_F), jnp.float32),
        ],
        compiler_params=pltpu.CompilerParams(
            dimension_semantics=("arbitrary",)),
    )(degp, xpad)


def _run_chunks(nchunks, gstart, sstart, gwait, swait):
    kmax = nchunks // 4
    gstart(0, 0)
    gstart(1, 1)
    gstart(2, 2)

    def body(kk, _):
        for b in range(4):
            j = 4 * kk + b
            gwait(b)
            sstart(j, b)
            tgt = (b + 3) % 4
            if b == 0:
                @pl.when(kk > 0)
                def _():
                    swait(tgt)

                gstart(j + 3, tgt)
            else:
                swait(tgt)

                @pl.when(kk < kmax - 1)
                def _():
                    gstart(j + 3, tgt)
        return 0

    lax.fori_loop(0, kmax, body, 0)
    swait(3)


@functools.partial(
    pl.kernel,
    out_type=jax.ShapeDtypeStruct((NC, NPAD, IN_F), jnp.float32),
    mesh=_mesh,
    compiler_params=_sc_params,
    scratch_types=[
        pltpu.VMEM((CH1, CHUNK), jnp.int32),
        pltpu.VMEM((CH1, CHUNK), jnp.int32),
        pltpu.VMEM((4, CHUNK, IN_F), jnp.float32),
        pltpu.VMEM((CHUNK, IN_F), jnp.float32),
        pltpu.VMEM_SHARED((NPAD, IN_F), jnp.float32),
        pltpu.SemaphoreType.DMA((4,)),
        pltpu.SemaphoreType.DMA((4,)),
    ],
)
def _pass1_kernel(src_hbm, dst_hbm, xs_hbm, acc_out,
                  sidx, didx, gbuf, zbuf, acc_sh, gsem, ssem):
    c = lax.axis_index("c")
    s = lax.axis_index("s")
    base = c * (ECH // NC) + s * CH1

    pltpu.sync_copy(src_hbm.at[pl.ds(base, CH1)], sidx)
    pltpu.sync_copy(dst_hbm.at[pl.ds(base, CH1)], didx)

    zeros = jnp.zeros((L,), jnp.float32)

    def zbody(i, _):
        zbuf[i, pl.ds(0, L)] = zeros
        return 0

    lax.fori_loop(0, CHUNK, zbody, 0)
    row0 = s * RPT
    for k in range(RFULL):
        pltpu.sync_copy(zbuf, acc_sh.at[pl.ds(row0 + k * CHUNK, CHUNK)])
    pltpu.sync_copy(zbuf.at[pl.ds(0, RREM)],
                    acc_sh.at[pl.ds(row0 + RFULL * CHUNK, RREM)])
    plsc.subcore_barrier()

    def gstart(j, b):
        pltpu.make_async_copy(
            xs_hbm.at[sidx.at[j]], gbuf.at[b], gsem.at[b]).start()

    def gwait(b):
        pltpu.make_async_copy(
            xs_hbm.at[sidx.at[0]], gbuf.at[b], gsem.at[b]).wait()

    def sstart(j, b):
        pltpu.make_async_copy(
            gbuf.at[b], acc_sh.at[didx.at[j]], ssem.at[b]).start(add=True)

    def swait(b):
        pltpu.make_async_copy(
            gbuf.at[b], acc_sh.at[didx.at[0]], ssem.at[b]).wait()

    _run_chunks(CH1, gstart, sstart, gwait, swait)
    plsc.subcore_barrier()

    for k in range(RFULL):
        pltpu.sync_copy(acc_sh.at[pl.ds(row0 + k * CHUNK, CHUNK)], zbuf)
        pltpu.sync_copy(zbuf, acc_out.at[c, pl.ds(row0 + k * CHUNK, CHUNK)])
    pltpu.sync_copy(acc_sh.at[pl.ds(row0 + RFULL * CHUNK, RREM)],
                    zbuf.at[pl.ds(0, RREM)])
    pltpu.sync_copy(zbuf.at[pl.ds(0, RREM)],
                    acc_out.at[c, pl.ds(row0 + RFULL * CHUNK, RREM)])


def _mid_body(acc1_ref, xs_ref, dinv_ref, w1_ref, b1_ref, hs_ref):
    a = acc1_ref[0] + acc1_ref[1] + xs_ref[...]
    dv = dinv_ref[...]
    y1 = a * dv
    h = jnp.dot(y1, w1_ref[...], preferred_element_type=jnp.float32)
    h = jnp.maximum(h + b1_ref[...], 0.0)
    hs = h * dv
    hs_ref[0] = hs[:, : H1 // 2]
    hs_ref[1] = hs[:, H1 // 2:]


def _mid(acc1, xs, dinv2d, W1, b1):
    return pl.pallas_call(
        _mid_body,
        grid=(TCG,),
        in_specs=[
            pl.BlockSpec((NC, TCB, IN_F), lambda i: (0, i, 0)),
            pl.BlockSpec((TCB, IN_F), lambda i: (i, 0)),
            pl.BlockSpec((TCB, 1), lambda i: (i, 0)),
            pl.BlockSpec((IN_F, H1), lambda i: (0, 0)),
            pl.BlockSpec((1, H1), lambda i: (0, 0)),
        ],
        out_specs=pl.BlockSpec((NC, TCB, H1 // 2), lambda i: (0, i, 0)),
        out_shape=jax.ShapeDtypeStruct((NC, NPAD, H1 // 2), jnp.float32),
        compiler_params=pltpu.CompilerParams(
            dimension_semantics=("arbitrary",)),
    )(acc1, xs, dinv2d, W1, b1)


@functools.partial(
    pl.kernel,
    out_type=jax.ShapeDtypeStruct((NC, NPAD, H1 // 2), jnp.float32),
    mesh=_mesh,
    compiler_params=_sc_params,
    scratch_types=[
        pltpu.VMEM((BLK2, CHUNK), jnp.int32),
        pltpu.VMEM((BLK2, CHUNK), jnp.int32),
        pltpu.VMEM((4, CHUNK, H1 // 2), jnp.float32),
        pltpu.VMEM((CHUNK, H1 // 2), jnp.float32),
        pltpu.VMEM_SHARED((NPAD, H1 // 2), jnp.float32),
        pltpu.SemaphoreType.DMA((4,)),
        pltpu.SemaphoreType.DMA((4,)),
    ],
)
def _pass2_kernel(src_hbm, dst_hbm, hs_hbm, acc_out,
                  sidx, didx, gbuf, zbuf, acc_sh, gsem, ssem):
    c = lax.axis_index("c")
    s = lax.axis_index("s")
    base = s * CH2

    zeros = jnp.zeros((L,), jnp.float32)

    def zbody(i, _):
        zbuf[i, pl.ds(0, L)] = zeros
        zbuf[i, pl.ds(L, L)] = zeros
        return 0

    lax.fori_loop(0, CHUNK, zbody, 0)
    row0 = s * RPT
    for k in range(RFULL):
        pltpu.sync_copy(zbuf, acc_sh.at[pl.ds(row0 + k * CHUNK, CHUNK)])
    pltpu.sync_copy(zbuf.at[pl.ds(0, RREM)],
                    acc_sh.at[pl.ds(row0 + RFULL * CHUNK, RREM)])
    plsc.subcore_barrier()

    def gstart(j, b):
        pltpu.make_async_copy(
            hs_hbm.at[c].at[sidx.at[j]], gbuf.at[b], gsem.at[b]).start()

    def gwait(b):
        pltpu.make_async_copy(
            hs_hbm.at[c].at[sidx.at[0]], gbuf.at[b], gsem.at[b]).wait()

    def sstart(j, b):
        pltpu.make_async_copy(
            gbuf.at[b], acc_sh.at[didx.at[j]], ssem.at[b]).start(add=True)

    def swait(b):
        pltpu.make_async_copy(
            gbuf.at[b], acc_sh.at[didx.at[0]], ssem.at[b]).wait()

    def blk_body(bb, _):
        pltpu.sync_copy(src_hbm.at[pl.ds(base + bb * BLK2, BLK2)], sidx)
        pltpu.sync_copy(dst_hbm.at[pl.ds(base + bb * BLK2, BLK2)], didx)
        _run_chunks(BLK2, gstart, sstart, gwait, swait)
        return 0

    lax.fori_loop(0, NB2, blk_body, 0)
    plsc.subcore_barrier()

    for k in range(RFULL):
        pltpu.sync_copy(acc_sh.at[pl.ds(row0 + k * CHUNK, CHUNK)], zbuf)
        pltpu.sync_copy(zbuf, acc_out.at[c, pl.ds(row0 + k * CHUNK, CHUNK)])
    pltpu.sync_copy(acc_sh.at[pl.ds(row0 + RFULL * CHUNK, RREM)],
                    zbuf.at[pl.ds(0, RREM)])
    pltpu.sync_copy(zbuf.at[pl.ds(0, RREM)],
                    acc_out.at[c, pl.ds(row0 + RFULL * CHUNK, RREM)])


def _out_body(acc2_ref, hs_ref, dinv_ref, w2_ref, wl_ref, b2_ref, bl_ref,
              out_ref):
    dv = dinv_ref[...]
    y = jnp.concatenate(
        [acc2_ref[0] + hs_ref[0], acc2_ref[1] + hs_ref[1]], axis=1)
    y = y * dv
    wf = jnp.dot(w2_ref[...], wl_ref[...],
                 preferred_element_type=jnp.float32)
    bf = jnp.dot(b2_ref[...], wl_ref[...],
                 preferred_element_type=jnp.float32) + bl_ref[...]
    out_ref[...] = jnp.dot(y, wf, preferred_element_type=jnp.float32) + bf


def _final(acc2, hs, dinv2d, W2, Wl, b2, bl):
    return pl.pallas_call(
        _out_body,
        grid=(TCG,),
        in_specs=[
            pl.BlockSpec((NC, TCB, H1 // 2), lambda i: (0, i, 0)),
            pl.BlockSpec((NC, TCB, H1 // 2), lambda i: (0, i, 0)),
            pl.BlockSpec((TCB, 1), lambda i: (i, 0)),
            pl.BlockSpec((H1, H2), lambda i: (0, 0)),
            pl.BlockSpec((H2, OUT_F), lambda i: (0, 0)),
            pl.BlockSpec((1, H2), lambda i: (0, 0)),
            pl.BlockSpec((1, OUT_F), lambda i: (0, 0)),
        ],
        out_specs=pl.BlockSpec((TCB, OUT_F), lambda i: (i, 0)),
        out_shape=jax.ShapeDtypeStruct((N, OUT_F), jnp.float32),
        compiler_params=pltpu.CompilerParams(
            dimension_semantics=("arbitrary",)),
    )(acc2, hs, dinv2d, W2, Wl, b2, bl)


def kernel(node_matrix, graph, W1, b1, W2, b2, Wl, bl):
    E = graph.shape[1]
    src = graph[0]
    dst = graph[1]
    sent = N + jnp.arange(E_PAD - E, dtype=jnp.int32) % (NPAD - N)
    dstp = jnp.concatenate([dst, sent]).reshape(ECH, CHUNK)
    srcp = jnp.concatenate([src, sent]).reshape(ECH, CHUNK)

    xpad = jnp.zeros((NPAD, IN_F), jnp.float32).at[:N].set(node_matrix)

    degp = _deg_kernel(dstp)
    dinv2d, xs = _dinv_xs(degp, xpad)
    acc1 = _pass1_kernel(srcp, dstp, xs)
    hs = _mid(acc1, xs, dinv2d, W1, b1.reshape(1, H1))
    acc2 = _pass2_kernel(srcp, dstp, hs)
    outp = _final(acc2, hs, dinv2d, W2, Wl,
                  b2.reshape(1, H2), bl.reshape(1, OUT_F))
    return outp
